# Initial kernel scaffold; baseline (speedup 1.0000x reference)
import jax
import jax.numpy as jnp
from jax.experimental import pallas as pl


def kernel(ids, table, W, b, gamma, beta):
    raise NotImplementedError("write your pallas kernel here")


# speedup vs baseline: 3.1177x; 3.1177x over previous
import functools

import jax
import jax.numpy as jnp
from jax import lax
from jax.experimental import pallas as pl
from jax.experimental.pallas import tpu as pltpu
from jax.experimental.pallas import tpu_sc as plsc


def _transform_body(table_ref, w_ref, b_ref, gamma_ref, beta_ref, out_ref):
    x = table_ref[...]
    h = lax.dot_general(
        x, w_ref[...], (((1,), (1,)), ((), ())),
        preferred_element_type=jnp.float32,
        precision=lax.Precision.HIGHEST,
    )
    h = h + b_ref[...]
    mean = jnp.mean(h, axis=1, keepdims=True)
    hc = h - mean
    var = jnp.mean(hc * hc, axis=1, keepdims=True)
    h = hc * lax.rsqrt(var + 1e-5) * gamma_ref[...] + beta_ref[...]
    norm2 = jnp.sum(h * h, axis=1, keepdims=True)
    out_ref[...] = h * lax.rsqrt(jnp.maximum(norm2, 1e-24))


def _transform_table(table, W, b, gamma, beta):
    n, d = table.shape
    return pl.pallas_call(
        _transform_body,
        out_shape=jax.ShapeDtypeStruct((n, d), jnp.float32),
    )(table, W, b.reshape(1, d), gamma.reshape(1, d), beta.reshape(1, d))


def _make_gather(b_total, d):
    info = plsc.get_sparse_core_info()
    nw = info.num_cores * info.num_subcores
    b_per_w = b_total // nw
    chunk = 128
    n_chunks = b_per_w // chunk
    mesh = plsc.VectorSubcoreMesh(core_axis_name="c", subcore_axis_name="s")

    @functools.partial(
        pl.kernel,
        out_type=jax.ShapeDtypeStruct((b_total, d), jnp.float32),
        mesh=mesh,
        scratch_types=[
            pltpu.VMEM((chunk,), jnp.int32),
            pltpu.VMEM((chunk, d), jnp.float32),
            pltpu.SemaphoreType.DMA,
        ],
    )
    def gather_k(tab_hbm, idx_hbm, out_hbm, idx_v, rows_v, sem):
        wid = lax.axis_index("s") * info.num_cores + lax.axis_index("c")
        base = wid * b_per_w
        for c in range(n_chunks):
            off = base + c * chunk
            pltpu.sync_copy(idx_hbm.at[pl.ds(off, chunk)], idx_v)
            pltpu.async_copy(tab_hbm.at[idx_v], rows_v, sem).wait()
            pltpu.sync_copy(rows_v, out_hbm.at[pl.ds(off, chunk)])

    return gather_k


def kernel(ids, table, W, b, gamma, beta):
    y_table = _transform_table(table, W, b, gamma, beta)
    gather_k = _make_gather(ids.shape[0], table.shape[1])
    return gather_k(y_table, ids.astype(jnp.int32))


# speedup vs baseline: 3.1550x; 1.0120x over previous
import functools

import jax
import jax.numpy as jnp
from jax import lax
from jax.experimental import pallas as pl
from jax.experimental.pallas import tpu as pltpu
from jax.experimental.pallas import tpu_sc as plsc


def _transform_body(table_ref, w_ref, b_ref, gamma_ref, beta_ref, out_ref):
    x = table_ref[...]
    h = lax.dot_general(
        x, w_ref[...], (((1,), (1,)), ((), ())),
        preferred_element_type=jnp.float32,
        precision=lax.Precision.HIGHEST,
    )
    h = h + b_ref[...]
    mean = jnp.mean(h, axis=1, keepdims=True)
    hc = h - mean
    var = jnp.mean(hc * hc, axis=1, keepdims=True)
    h = hc * lax.rsqrt(var + 1e-5) * gamma_ref[...] + beta_ref[...]
    norm2 = jnp.sum(h * h, axis=1, keepdims=True)
    out_ref[...] = h * lax.rsqrt(jnp.maximum(norm2, 1e-24))


def _transform_table(table, W, b, gamma, beta):
    n, d = table.shape
    return pl.pallas_call(
        _transform_body,
        out_shape=jax.ShapeDtypeStruct((n, d), jnp.float32),
    )(table, W, b.reshape(1, d), gamma.reshape(1, d), beta.reshape(1, d))


def _make_gather(b_total, d):
    info = plsc.get_sparse_core_info()
    nw = info.num_cores * info.num_subcores
    b_per_w = b_total // nw
    chunk = 64
    n_chunks = b_per_w // chunk
    mesh = plsc.VectorSubcoreMesh(core_axis_name="c", subcore_axis_name="s")

    @functools.partial(
        pl.kernel,
        out_type=jax.ShapeDtypeStruct((b_total, d), jnp.float32),
        mesh=mesh,
        scratch_types=[
            pltpu.VMEM((b_per_w,), jnp.int32),
            pltpu.VMEM((chunk, d), jnp.float32),
            pltpu.VMEM((chunk, d), jnp.float32),
            pltpu.SemaphoreType.DMA,
            pltpu.SemaphoreType.DMA,
            pltpu.SemaphoreType.DMA,
            pltpu.SemaphoreType.DMA,
        ],
    )
    def gather_k(tab_hbm, idx_hbm, out_hbm, idx_v, rows0, rows1,
                 gsem0, gsem1, ssem0, ssem1):
        wid = lax.axis_index("s") * info.num_cores + lax.axis_index("c")
        base = wid * b_per_w
        bufs = (rows0, rows1)
        gsems = (gsem0, gsem1)
        ssems = (ssem0, ssem1)
        pltpu.sync_copy(idx_hbm.at[pl.ds(base, b_per_w)], idx_v)
        gat = [None, None]
        sto = [None, None]
        for c in range(n_chunks + 1):
            if c < n_chunks:
                i = c % 2
                if sto[i] is not None:
                    sto[i].wait()
                gat[i] = pltpu.async_copy(
                    tab_hbm.at[idx_v.at[pl.ds(c * chunk, chunk)]],
                    bufs[i], gsems[i])
            if c >= 1:
                j = (c - 1) % 2
                gat[j].wait()
                sto[j] = pltpu.async_copy(
                    bufs[j], out_hbm.at[pl.ds(base + (c - 1) * chunk, chunk)],
                    ssems[j])
        sto[0].wait()
        sto[1].wait()

    return gather_k


def kernel(ids, table, W, b, gamma, beta):
    y_table = _transform_table(table, W, b, gamma, beta)
    gather_k = _make_gather(ids.shape[0], table.shape[1])
    return gather_k(y_table, ids.astype(jnp.int32))


# speedup vs baseline: 3.3309x; 1.0558x over previous
import functools

import jax
import jax.numpy as jnp
from jax import lax
from jax.experimental import pallas as pl
from jax.experimental.pallas import tpu as pltpu
from jax.experimental.pallas import tpu_sc as plsc


def _transform_body(table_ref, w_ref, b_ref, gamma_ref, beta_ref, out_ref):
    x = table_ref[...]
    h = lax.dot_general(
        x, w_ref[...], (((1,), (1,)), ((), ())),
        preferred_element_type=jnp.float32,
    )
    h = h + b_ref[...]
    mean = jnp.mean(h, axis=1, keepdims=True)
    hc = h - mean
    var = jnp.mean(hc * hc, axis=1, keepdims=True)
    h = hc * lax.rsqrt(var + 1e-5) * gamma_ref[...] + beta_ref[...]
    norm2 = jnp.sum(h * h, axis=1, keepdims=True)
    out_ref[...] = h * lax.rsqrt(jnp.maximum(norm2, 1e-24))


def _transform_table(table, W, b, gamma, beta):
    n, d = table.shape
    return pl.pallas_call(
        _transform_body,
        out_shape=jax.ShapeDtypeStruct((n, d), jnp.float32),
    )(table, W, b.reshape(1, d), gamma.reshape(1, d), beta.reshape(1, d))


def _make_gather(b_total, d):
    info = plsc.get_sparse_core_info()
    nw = info.num_cores * info.num_subcores
    b_per_w = b_total // nw
    chunk = 64
    n_chunks = b_per_w // chunk
    mesh = plsc.VectorSubcoreMesh(core_axis_name="c", subcore_axis_name="s")

    @functools.partial(
        pl.kernel,
        out_type=jax.ShapeDtypeStruct((b_total, d), jnp.float32),
        mesh=mesh,
        scratch_types=[
            pltpu.VMEM((b_per_w,), jnp.int32),
            pltpu.VMEM((chunk, d), jnp.float32),
            pltpu.VMEM((chunk, d), jnp.float32),
            pltpu.SemaphoreType.DMA,
            pltpu.SemaphoreType.DMA,
            pltpu.SemaphoreType.DMA,
            pltpu.SemaphoreType.DMA,
        ],
    )
    def gather_k(tab_hbm, idx_hbm, out_hbm, idx_v, rows0, rows1,
                 gsem0, gsem1, ssem0, ssem1):
        wid = lax.axis_index("s") * info.num_cores + lax.axis_index("c")
        base = wid * b_per_w
        bufs = (rows0, rows1)
        gsems = (gsem0, gsem1)
        ssems = (ssem0, ssem1)
        pltpu.sync_copy(idx_hbm.at[pl.ds(base, b_per_w)], idx_v)
        gat = [None, None]
        sto = [None, None]
        for c in range(n_chunks + 1):
            if c < n_chunks:
                i = c % 2
                if sto[i] is not None:
                    sto[i].wait()
                gat[i] = pltpu.async_copy(
                    tab_hbm.at[idx_v.at[pl.ds(c * chunk, chunk)]],
                    bufs[i], gsems[i])
            if c >= 1:
                j = (c - 1) % 2
                gat[j].wait()
                sto[j] = pltpu.async_copy(
                    bufs[j], out_hbm.at[pl.ds(base + (c - 1) * chunk, chunk)],
                    ssems[j])
        sto[0].wait()
        sto[1].wait()

    return gather_k


def kernel(ids, table, W, b, gamma, beta):
    y_table = _transform_table(table, W, b, gamma, beta)
    gather_k = _make_gather(ids.shape[0], table.shape[1])
    return gather_k(y_table, ids.astype(jnp.int32))
